# baseline (speedup 1.0000x reference)
import functools

import jax
import jax.numpy as jnp
from jax import lax
from jax.experimental import pallas as pl
from jax.experimental.pallas import tpu as pltpu
from jax.experimental.pallas import tpu_sc as plsc

NUM_E = 100000
NUM_R = 1000
DIM = 128
BATCH = 16384

L = 16
NW = 32
B_PER_W = BATCH // NW
CHUNK = 128
NCHUNK = B_PER_W // CHUNK

_C1 = -0.31753146
_C2 = 0.0798171


def _rsqrt(x):
    i = plsc.bitcast(x, jnp.int32)
    i = jnp.int32(0x5F3759DF) - (i >> 1)
    y = plsc.bitcast(i, jnp.float32)
    for _ in range(2):
        y = y * (1.5 - 0.5 * x * y * y)
    return y


def _body(src_hbm, pred_hbm, tail_hbm, ev_hbm, er_hbm, out_hbm,
          idx_s, idx_p, idx_t, s0, t0, r0, s1, t1, r1, out_buf,
          sem0, sem1, sem2, sem3):
    wid = lax.axis_index("s") * 2 + lax.axis_index("c")
    base = wid * B_PER_W

    isl = pl.ds(base, B_PER_W)
    cp_is = pltpu.make_async_copy(src_hbm.at[isl], idx_s, sem0)
    cp_ip = pltpu.make_async_copy(pred_hbm.at[isl], idx_p, sem0)
    cp_it = pltpu.make_async_copy(tail_hbm.at[isl], idx_t, sem0)
    cp_is.start()
    cp_ip.start()
    cp_it.start()
    cp_is.wait()
    cp_ip.wait()
    cp_it.wait()

    bufs = ((s0, t0, r0, sem0), (s1, t1, r1, sem1))

    def fire(c):
        sb, tb, rb, sem = bufs[c % 2]
        sl = pl.ds(c * CHUNK, CHUNK)
        cps = (
            pltpu.make_async_copy(ev_hbm.at[idx_s.at[sl]], sb, sem),
            pltpu.make_async_copy(ev_hbm.at[idx_t.at[sl]], tb, sem),
            pltpu.make_async_copy(er_hbm.at[idx_p.at[sl]], rb, sem),
        )
        for cp in cps:
            cp.start()
        return cps

    lane = jnp.arange(L, dtype=jnp.int32)
    zero = jnp.zeros((L,), jnp.float32)

    half = CHUNK // 2

    def fire_half(lo, sem):
        sl = pl.ds(lo, half)
        dst = pl.ds(lo, half)
        cps = (
            pltpu.make_async_copy(ev_hbm.at[idx_s.at[sl]], s0.at[dst], sem),
            pltpu.make_async_copy(ev_hbm.at[idx_t.at[sl]], t0.at[dst], sem),
            pltpu.make_async_copy(er_hbm.at[idx_p.at[sl]], r0.at[dst], sem),
        )
        for cp in cps:
            cp.start()
        return cps

    pend_a = fire_half(0, sem0)
    pend_b = fire_half(half, sem2)
    pend = fire(1)
    halves = ((pend_a, 0, CHUNK // L // 2), (pend_b, CHUNK // L // 2, CHUNK // L))

    for c in range(NCHUNK):
        s_buf, t_buf, r_buf, _ = bufs[c % 2]
        if c == 0:
            spans = halves
        else:
            for cp in pend:
                cp.wait()
            if c + 1 < NCHUNK:
                pend = fire(c + 1)
            spans = ((None, 0, CHUNK // L),)

        def grp(g, _, s_buf=s_buf, t_buf=t_buf, r_buf=r_buf, c=c):
            rows = g * L + lane

            def pass1(d0, carry):
                ssa, sta = carry
                cb = (d0 + lane) & (L - 1)
                for m in range(DIM // L):
                    col = cb + (L * m)
                    s = plsc.load_gather(s_buf, [rows, col])
                    t = plsc.load_gather(t_buf, [rows, col])
                    ssa = ssa + s * s
                    sta = sta + t * t
                return ssa, sta

            ss_s, ss_t = lax.fori_loop(0, L, pass1, (zero, zero),
                                       unroll=2)

            ist = _rsqrt(ss_s * ss_t)
            i2 = ist * ist
            a_s = i2 * ss_t
            a_t = i2 * ss_s
            c1s = _C1 * a_s
            c2s = _C2 * (a_s * a_s)
            c1t = _C1 * a_t
            c2t = _C2 * (a_t * a_t)
            pk = lambda a, b: plsc.pack(a, b, format=plsc.PackFormat.INTERLEAVED)
            c1sp = pk(c1s, c1s)
            c2sp = pk(c2s, c2s)
            c1tp = pk(c1t, c1t)
            c2tp = pk(c2t, c2t)

            def pass2(d0, acc):
                cb = (d0 + lane) & (L - 1)
                for m in range(0, DIM // L, 2):
                    col_a = cb + (L * m)
                    col_b = cb + (L * (m + 1))
                    sa = plsc.load_gather(s_buf, [rows, col_a])
                    sb = plsc.load_gather(s_buf, [rows, col_b])
                    ta = plsc.load_gather(t_buf, [rows, col_a])
                    tb = plsc.load_gather(t_buf, [rows, col_b])
                    ra = plsc.load_gather(r_buf, [rows, col_a])
                    rb = plsc.load_gather(r_buf, [rows, col_b])
                    sp = pk(sa, sb)
                    tp = pk(ta, tb)
                    rp = pk(ra, rb)
                    ws = sp * sp
                    wt = tp * tp
                    hs = (c2sp * ws + c1sp) * ws + 1.0
                    ht = (c2tp * wt + c1tp) * wt + 1.0
                    p = (sp * tp) * (hs * ht) * rp
                    pa, pb = plsc.unpack(p, format=plsc.PackFormat.INTERLEAVED)
                    acc = acc + pa + pb
                return acc

            acc = lax.fori_loop(0, L, pass2, zero)
            out_buf[pl.ds(c * CHUNK + g * L, L)] = acc * ist
            return 0

        for cps, g_lo, g_hi in spans:
            if cps is not None:
                for cp in cps:
                    cp.wait()
            lax.fori_loop(g_lo, g_hi, grp, 0)

        pltpu.make_async_copy(
            out_buf.at[pl.ds(c * CHUNK, CHUNK)],
            out_hbm.at[pl.ds(base + c * CHUNK, CHUNK)],
            sem3,
        ).start()

    for c in range(NCHUNK):
        pltpu.make_async_copy(
            out_buf.at[pl.ds(c * CHUNK, CHUNK)],
            out_hbm.at[pl.ds(base + c * CHUNK, CHUNK)],
            sem3,
        ).wait()


@functools.partial(
    pl.kernel,
    mesh=plsc.VectorSubcoreMesh(core_axis_name="c", subcore_axis_name="s"),
    out_type=jax.ShapeDtypeStruct((BATCH,), jnp.float32),
    compiler_params=pltpu.CompilerParams(needs_layout_passes=False),
    scratch_types=[
        pltpu.VMEM((B_PER_W,), jnp.int32),
        pltpu.VMEM((B_PER_W,), jnp.int32),
        pltpu.VMEM((B_PER_W,), jnp.int32),
        pltpu.VMEM((CHUNK, DIM), jnp.float32),
        pltpu.VMEM((CHUNK, DIM), jnp.float32),
        pltpu.VMEM((CHUNK, DIM), jnp.float32),
        pltpu.VMEM((CHUNK, DIM), jnp.float32),
        pltpu.VMEM((CHUNK, DIM), jnp.float32),
        pltpu.VMEM((CHUNK, DIM), jnp.float32),
        pltpu.VMEM((B_PER_W,), jnp.float32),
        pltpu.SemaphoreType.DMA,
        pltpu.SemaphoreType.DMA,
        pltpu.SemaphoreType.DMA,
        pltpu.SemaphoreType.DMA,
    ],
)
def _distmult_sc(src, pred, tail, ev, er, out, *scratch):
    _body(src, pred, tail, ev, er, out, *scratch)


@jax.jit
def kernel(src, pred, tail, E_v, E_r):
    out = _distmult_sc(
        src.astype(jnp.int32),
        pred.astype(jnp.int32),
        tail.astype(jnp.int32),
        E_v,
        E_r,
    )
    return out.reshape(BATCH, 1)


# speedup vs baseline: 1.0756x; 1.0756x over previous
import functools

import jax
import jax.numpy as jnp
from jax import lax
from jax.experimental import pallas as pl
from jax.experimental.pallas import tpu as pltpu
from jax.experimental.pallas import tpu_sc as plsc

NUM_E = 100000
NUM_R = 1000
DIM = 128
BATCH = 16384

L = 16
NW = 32
B_PER_W = BATCH // NW
CHUNK = 128
NCHUNK = B_PER_W // CHUNK

_C1 = -0.31753146
_C2 = 0.0798171


def _rsqrt(x):
    i = plsc.bitcast(x, jnp.int32)
    i = jnp.int32(0x5F3759DF) - (i >> 1)
    y = plsc.bitcast(i, jnp.float32)
    for _ in range(2):
        y = y * (1.5 - 0.5 * x * y * y)
    return y


def _body(src_hbm, pred_hbm, tail_hbm, ev_hbm, er_hbm, out_hbm,
          idx_s, idx_p, idx_t, s0, t0, r0, s1, t1, r1, out_buf,
          sem0, sem1):
    wid = lax.axis_index("s") * 2 + lax.axis_index("c")
    base = wid * B_PER_W

    isl = pl.ds(base, B_PER_W)
    cp_is = pltpu.make_async_copy(src_hbm.at[isl], idx_s, sem0)
    cp_ip = pltpu.make_async_copy(pred_hbm.at[isl], idx_p, sem0)
    cp_it = pltpu.make_async_copy(tail_hbm.at[isl], idx_t, sem0)
    cp_is.start()
    cp_ip.start()
    cp_it.start()
    cp_is.wait()
    cp_ip.wait()
    cp_it.wait()

    bufs = ((s0, t0, r0, sem0), (s1, t1, r1, sem1))

    def fire(c):
        sb, tb, rb, sem = bufs[c % 2]
        sl = pl.ds(c * CHUNK, CHUNK)
        cps = (
            pltpu.make_async_copy(ev_hbm.at[idx_s.at[sl]], sb, sem),
            pltpu.make_async_copy(ev_hbm.at[idx_t.at[sl]], tb, sem),
            pltpu.make_async_copy(er_hbm.at[idx_p.at[sl]], rb, sem),
        )
        for cp in cps:
            cp.start()
        return cps

    lane = jnp.arange(L, dtype=jnp.int32)
    zero = jnp.zeros((L,), jnp.float32)

    pend = fire(0)
    for c in range(NCHUNK):
        for cp in pend:
            cp.wait()
        s_buf, t_buf, r_buf, _ = bufs[c % 2]
        if c + 1 < NCHUNK:
            pend = fire(c + 1)

        def grp(g, _, s_buf=s_buf, t_buf=t_buf, r_buf=r_buf, c=c):
            rows = g * L + lane

            def pass1(d0, carry):
                ssa, sta = carry
                cb = (d0 + lane) & (L - 1)
                for m in range(DIM // L):
                    col = cb + (L * m)
                    s = plsc.load_gather(s_buf, [rows, col])
                    t = plsc.load_gather(t_buf, [rows, col])
                    ssa = ssa + s * s
                    sta = sta + t * t
                return ssa, sta

            ss_s, ss_t = lax.fori_loop(0, L, pass1, (zero, zero),
                                       unroll=2)

            ist = _rsqrt(ss_s * ss_t)
            i2 = ist * ist
            a_s = i2 * ss_t
            a_t = i2 * ss_s
            c1s = _C1 * a_s
            c2s = _C2 * (a_s * a_s)
            c1t = _C1 * a_t
            c2t = _C2 * (a_t * a_t)
            pk = lambda a, b: plsc.pack(a, b, format=plsc.PackFormat.INTERLEAVED)
            c1sp = pk(c1s, c1s)
            c2sp = pk(c2s, c2s)
            c1tp = pk(c1t, c1t)
            c2tp = pk(c2t, c2t)

            def pass2(d0, acc):
                cb = (d0 + lane) & (L - 1)
                for m in range(0, DIM // L, 2):
                    col_a = cb + (L * m)
                    col_b = cb + (L * (m + 1))
                    sa = plsc.load_gather(s_buf, [rows, col_a])
                    sb = plsc.load_gather(s_buf, [rows, col_b])
                    ta = plsc.load_gather(t_buf, [rows, col_a])
                    tb = plsc.load_gather(t_buf, [rows, col_b])
                    ra = plsc.load_gather(r_buf, [rows, col_a])
                    rb = plsc.load_gather(r_buf, [rows, col_b])
                    sp = pk(sa, sb)
                    tp = pk(ta, tb)
                    rp = pk(ra, rb)
                    ws = sp * sp
                    wt = tp * tp
                    hs = (c2sp * ws + c1sp) * ws + 1.0
                    ht = (c2tp * wt + c1tp) * wt + 1.0
                    p = (sp * tp) * (hs * ht) * rp
                    pa, pb = plsc.unpack(p, format=plsc.PackFormat.INTERLEAVED)
                    acc = acc + pa + pb
                return acc

            acc = lax.fori_loop(0, L, pass2, zero)
            out_buf[pl.ds(c * CHUNK + g * L, L)] = acc * ist
            return 0

        lax.fori_loop(0, CHUNK // L, grp, 0)

    pltpu.sync_copy(out_buf, out_hbm.at[pl.ds(base, B_PER_W)])


@functools.partial(
    pl.kernel,
    mesh=plsc.VectorSubcoreMesh(core_axis_name="c", subcore_axis_name="s"),
    out_type=jax.ShapeDtypeStruct((BATCH,), jnp.float32),
    compiler_params=pltpu.CompilerParams(needs_layout_passes=False),
    scratch_types=[
        pltpu.VMEM((B_PER_W,), jnp.int32),
        pltpu.VMEM((B_PER_W,), jnp.int32),
        pltpu.VMEM((B_PER_W,), jnp.int32),
        pltpu.VMEM((CHUNK, DIM), jnp.float32),
        pltpu.VMEM((CHUNK, DIM), jnp.float32),
        pltpu.VMEM((CHUNK, DIM), jnp.float32),
        pltpu.VMEM((CHUNK, DIM), jnp.float32),
        pltpu.VMEM((CHUNK, DIM), jnp.float32),
        pltpu.VMEM((CHUNK, DIM), jnp.float32),
        pltpu.VMEM((B_PER_W,), jnp.float32),
        pltpu.SemaphoreType.DMA,
        pltpu.SemaphoreType.DMA,
    ],
)
def _distmult_sc(src, pred, tail, ev, er, out, *scratch):
    _body(src, pred, tail, ev, er, out, *scratch)


@jax.jit
def kernel(src, pred, tail, E_v, E_r):
    out = _distmult_sc(
        src.astype(jnp.int32),
        pred.astype(jnp.int32),
        tail.astype(jnp.int32),
        E_v,
        E_r,
    )
    return out.reshape(BATCH, 1)


# speedup vs baseline: 1.0820x; 1.0059x over previous
import functools

import jax
import jax.numpy as jnp
from jax import lax
from jax.experimental import pallas as pl
from jax.experimental.pallas import tpu as pltpu
from jax.experimental.pallas import tpu_sc as plsc

NUM_E = 100000
NUM_R = 1000
DIM = 128
BATCH = 16384

L = 16
NW = 32
B_PER_W = BATCH // NW
CHUNK = 128
NCHUNK = B_PER_W // CHUNK

_C1 = -0.31753146
_C2 = 0.0798171


def _rsqrt(x):
    i = plsc.bitcast(x, jnp.int32)
    i = jnp.int32(0x5F3759DF) - (i >> 1)
    y = plsc.bitcast(i, jnp.float32)
    for _ in range(2):
        y = y * (1.5 - 0.5 * x * y * y)
    return y


def _body(src_hbm, pred_hbm, tail_hbm, ev_hbm, er_hbm, out_hbm,
          idx_s, idx_p, idx_t, s0, t0, r0, s1, t1, r1, out_buf,
          sem0, sem1):
    wid = lax.axis_index("s") * 2 + lax.axis_index("c")
    base = wid * B_PER_W

    isl = pl.ds(base, B_PER_W)
    cp_is = pltpu.make_async_copy(src_hbm.at[isl], idx_s, sem0)
    cp_ip = pltpu.make_async_copy(pred_hbm.at[isl], idx_p, sem0)
    cp_it = pltpu.make_async_copy(tail_hbm.at[isl], idx_t, sem0)
    cp_is.start()
    cp_ip.start()
    cp_it.start()
    cp_is.wait()
    cp_ip.wait()
    cp_it.wait()

    bufs = ((s0, t0, r0, sem0), (s1, t1, r1, sem1))

    def fire(c):
        sb, tb, rb, sem = bufs[c % 2]
        sl = pl.ds(c * CHUNK, CHUNK)
        cps = (
            pltpu.make_async_copy(ev_hbm.at[idx_s.at[sl]], sb, sem),
            pltpu.make_async_copy(ev_hbm.at[idx_t.at[sl]], tb, sem),
            pltpu.make_async_copy(er_hbm.at[idx_p.at[sl]], rb, sem),
        )
        for cp in cps:
            cp.start()
        return cps

    lane = jnp.arange(L, dtype=jnp.int32)
    zero = jnp.zeros((L,), jnp.float32)

    pend = fire(0)
    for c in range(NCHUNK):
        for cp in pend:
            cp.wait()
        s_buf, t_buf, r_buf, _ = bufs[c % 2]
        if c + 1 < NCHUNK:
            pend = fire(c + 1)

        def grp(g, _, s_buf=s_buf, t_buf=t_buf, r_buf=r_buf, c=c):
            rows = g * L + lane

            def pass1(d0, carry):
                ssa, sta = carry
                cb = (d0 + lane) & (L - 1)
                for m in range(DIM // L):
                    col = cb + (L * m)
                    s = plsc.load_gather(s_buf, [rows, col])
                    t = plsc.load_gather(t_buf, [rows, col])
                    ssa = ssa + s * s
                    sta = sta + t * t
                return ssa, sta

            ss_s, ss_t = lax.fori_loop(0, L, pass1, (zero, zero))

            ist = _rsqrt(ss_s * ss_t)
            i2 = ist * ist
            a_s = i2 * ss_t
            a_t = i2 * ss_s
            c1s = _C1 * a_s
            c2s = _C2 * (a_s * a_s)
            c1t = _C1 * a_t
            c2t = _C2 * (a_t * a_t)
            pk = lambda a, b: plsc.pack(a, b, format=plsc.PackFormat.INTERLEAVED)
            c1sp = pk(c1s, c1s)
            c2sp = pk(c2s, c2s)
            c1tp = pk(c1t, c1t)
            c2tp = pk(c2t, c2t)

            def pass2(d0, acc):
                cb = (d0 + lane) & (L - 1)
                for m in range(0, DIM // L, 2):
                    col_a = cb + (L * m)
                    col_b = cb + (L * (m + 1))
                    sa = plsc.load_gather(s_buf, [rows, col_a])
                    sb = plsc.load_gather(s_buf, [rows, col_b])
                    ta = plsc.load_gather(t_buf, [rows, col_a])
                    tb = plsc.load_gather(t_buf, [rows, col_b])
                    ra = plsc.load_gather(r_buf, [rows, col_a])
                    rb = plsc.load_gather(r_buf, [rows, col_b])
                    sp = pk(sa, sb)
                    tp = pk(ta, tb)
                    rp = pk(ra, rb)
                    ws = sp * sp
                    wt = tp * tp
                    hs = (c2sp * ws + c1sp) * ws + 1.0
                    ht = (c2tp * wt + c1tp) * wt + 1.0
                    p = (sp * tp) * (hs * ht) * rp
                    pa, pb = plsc.unpack(p, format=plsc.PackFormat.INTERLEAVED)
                    acc = acc + pa + pb
                return acc

            acc = lax.fori_loop(0, L, pass2, zero)
            out_buf[pl.ds(c * CHUNK + g * L, L)] = acc * ist
            return 0

        lax.fori_loop(0, CHUNK // L, grp, 0)

    pltpu.sync_copy(out_buf, out_hbm.at[pl.ds(base, B_PER_W)])


@functools.partial(
    pl.kernel,
    mesh=plsc.VectorSubcoreMesh(core_axis_name="c", subcore_axis_name="s"),
    out_type=jax.ShapeDtypeStruct((BATCH,), jnp.float32),
    compiler_params=pltpu.CompilerParams(needs_layout_passes=False),
    scratch_types=[
        pltpu.VMEM((B_PER_W,), jnp.int32),
        pltpu.VMEM((B_PER_W,), jnp.int32),
        pltpu.VMEM((B_PER_W,), jnp.int32),
        pltpu.VMEM((CHUNK, DIM), jnp.float32),
        pltpu.VMEM((CHUNK, DIM), jnp.float32),
        pltpu.VMEM((CHUNK, DIM), jnp.float32),
        pltpu.VMEM((CHUNK, DIM), jnp.float32),
        pltpu.VMEM((CHUNK, DIM), jnp.float32),
        pltpu.VMEM((CHUNK, DIM), jnp.float32),
        pltpu.VMEM((B_PER_W,), jnp.float32),
        pltpu.SemaphoreType.DMA,
        pltpu.SemaphoreType.DMA,
    ],
)
def _distmult_sc(src, pred, tail, ev, er, out, *scratch):
    _body(src, pred, tail, ev, er, out, *scratch)


@jax.jit
def kernel(src, pred, tail, E_v, E_r):
    out = _distmult_sc(
        src.astype(jnp.int32),
        pred.astype(jnp.int32),
        tail.astype(jnp.int32),
        E_v,
        E_r,
    )
    return out.reshape(BATCH, 1)
